# Initial kernel scaffold; baseline (speedup 1.0000x reference)
import jax
import jax.numpy as jnp
from jax import lax
from jax.experimental import pallas as pl
from jax.experimental.pallas import tpu as pltpu
from jax.experimental.pallas import tpu_sc as plsc

SEQ = 200
BATCH = 4096
EMB = 32
NC = 2
NS = 16
NW = NC * NS
BPW = BATCH // NW
CH = 4
NSLOT = 2 * CH


def _danmodel_body(inp_hbm, tbl_hbm, wb_hbm, out_hbm,
                   idx_v, rows_v, acc_v, wb_v, out_v, sem_a, sem_b):
  wid = lax.axis_index("s") * NC + lax.axis_index("c")
  base = wid * BPW

  pltpu.sync_copy(inp_hbm.at[:, pl.ds(base, BPW)], idx_v)
  pltpu.sync_copy(wb_hbm, wb_v)

  zeros = jnp.zeros((16,), jnp.float32)

  @plsc.parallel_loop(0, BPW, 1, unroll=4)
  def _(b):
    acc_v[b, pl.ds(0, 16)] = zeros
    acc_v[b, pl.ds(16, 16)] = zeros

  def start_group(s0, c0, sem):
    for k in range(CH):
      pltpu.async_copy(
          tbl_hbm.at[idx_v.at[s0 + k]],
          rows_v.at[pl.ds((c0 + k) * BPW, BPW), :],
          sem)

  def drain_group(c0, sem):
    for k in range(CH):
      pltpu.make_async_copy(
          tbl_hbm.at[idx_v.at[0]],
          rows_v.at[pl.ds((c0 + k) * BPW, BPW), :],
          sem).wait()

  def accum_group(c0):
    @plsc.parallel_loop(0, BPW, 1, unroll=2)
    def _(b):
      for h in range(2):
        sl = pl.ds(h * 16, 16)
        v01 = rows_v[(c0 + 0) * BPW + b, sl] + rows_v[(c0 + 1) * BPW + b, sl]
        v23 = rows_v[(c0 + 2) * BPW + b, sl] + rows_v[(c0 + 3) * BPW + b, sl]
        plsc.addupdate(acc_v.at[b, sl], v01 + v23)

  start_group(0, 0, sem_a)

  def body(i, _):
    s0 = (2 * CH) * i
    drain_group(0, sem_a)
    start_group(s0 + CH, CH, sem_b)
    accum_group(0)
    drain_group(CH, sem_b)

    @pl.when(s0 + 2 * CH < SEQ)
    def _():
      start_group(s0 + 2 * CH, 0, sem_a)

    accum_group(CH)
    return 0

  lax.fori_loop(0, SEQ // (2 * CH), body, 0)

  inv_seq = jnp.float32(1.0 / SEQ)
  bias = wb_v[EMB]
  lanes = jnp.arange(16, dtype=jnp.int32)

  def ep_body(g, _):
    idxb = g * 16 + lanes
    o = jnp.full((16,), bias, jnp.float32)
    for d in range(EMB):
      col = plsc.load_gather(acc_v, [idxb, jnp.full((16,), d, jnp.int32)])
      x = col * inv_seq
      e = jnp.exp(jnp.abs(x) * 2.0)
      t = jnp.sign(x) * (1.0 - 2.0 / (e + 1.0))
      o = o + t * wb_v[d]
    out_v[pl.ds(g * 16, 16)] = o
    return 0

  lax.fori_loop(0, BPW // 16, ep_body, 0)

  pltpu.sync_copy(out_v, out_hbm.at[pl.ds(base, BPW)])


@jax.jit
def _danmodel(inp, tbl, wb):
  mesh = plsc.VectorSubcoreMesh(
      core_axis_name="c", subcore_axis_name="s", num_cores=NC,
      num_subcores=NS)
  return pl.kernel(
      _danmodel_body,
      out_type=jax.ShapeDtypeStruct((BATCH,), jnp.float32),
      mesh=mesh,
      scratch_types=[
          pltpu.VMEM((SEQ, BPW), jnp.int32),
          pltpu.VMEM((NSLOT * BPW, EMB), jnp.float32),
          pltpu.VMEM((BPW, EMB), jnp.float32),
          pltpu.VMEM((64,), jnp.float32),
          pltpu.VMEM((BPW,), jnp.float32),
          pltpu.SemaphoreType.DMA,
          pltpu.SemaphoreType.DMA,
      ],
  )(inp, tbl, wb)


def kernel(input, emb_table, W, b):
  inp = input.astype(jnp.int32)
  wb = jnp.concatenate(
      [W.reshape(-1), b.reshape(-1),
       jnp.zeros((64 - EMB - 1,), jnp.float32)]).astype(jnp.float32)
  out = _danmodel(inp, emb_table, wb)
  return out.reshape(BATCH, 1)


# speedup vs baseline: 1.9158x; 1.9158x over previous
import jax
import jax.numpy as jnp
from jax import lax
from jax.experimental import pallas as pl
from jax.experimental.pallas import tpu as pltpu
from jax.experimental.pallas import tpu_sc as plsc

SEQ = 200
BATCH = 4096
EMB = 32
NC = 2
NS = 16
NW = NC * NS
BPW = BATCH // NW
CH = 4
NSLOT = 2 * CH


def _danmodel_body(inp_hbm, tbl_hbm, wb_hbm, out_hbm,
                   idx_v, rows_v, acc_v, wb_v, out_v, sem_a, sem_b):
  wid = lax.axis_index("s") * NC + lax.axis_index("c")
  base = wid * BPW

  pltpu.sync_copy(inp_hbm.at[:, pl.ds(base, BPW)], idx_v)
  pltpu.sync_copy(wb_hbm, wb_v)

  zeros = jnp.zeros((16,), jnp.float32)

  @plsc.parallel_loop(0, BPW, 1, unroll=4)
  def _(b):
    acc_v[b, pl.ds(0, 16)] = zeros
    acc_v[b, pl.ds(16, 16)] = zeros

  def start_group(s0, c0, sem):
    for k in range(CH):
      pltpu.async_copy(
          tbl_hbm.at[idx_v.at[s0 + k]],
          rows_v.at[pl.ds((c0 + k) * BPW, BPW), :],
          sem)

  def drain_group(c0, sem):
    for k in range(CH):
      pltpu.make_async_copy(
          tbl_hbm.at[idx_v.at[0]],
          rows_v.at[pl.ds((c0 + k) * BPW, BPW), :],
          sem).wait()

  def accum_group(c0):
    @plsc.parallel_loop(0, BPW, 1, unroll=2)
    def _(b):
      for h in range(2):
        sl = pl.ds(h * 16, 16)
        v01 = rows_v[(c0 + 0) * BPW + b, sl] + rows_v[(c0 + 1) * BPW + b, sl]
        v23 = rows_v[(c0 + 2) * BPW + b, sl] + rows_v[(c0 + 3) * BPW + b, sl]
        plsc.addupdate(acc_v.at[b, sl], v01 + v23)

  start_group(0, 0, sem_a)

  def body(i, _):
    s0 = (2 * CH) * i
    drain_group(0, sem_a)
    start_group(s0 + CH, CH, sem_b)
    accum_group(0)
    drain_group(CH, sem_b)

    @pl.when(s0 + 2 * CH < SEQ)
    def _():
      start_group(s0 + 2 * CH, 0, sem_a)

    accum_group(CH)
    return 0

  lax.fori_loop(0, SEQ // (2 * CH), body, 0)

  inv_seq = jnp.float32(1.0 / SEQ)
  w_lo = wb_v[pl.ds(0, 16)]
  w_hi = wb_v[pl.ds(16, 16)]
  bias = wb_v[pl.ds(EMB, 16)][0]
  lanes = jnp.arange(16, dtype=jnp.int32)

  def ep_body(g, _):
    idxb = g * 16 + lanes
    o = jnp.full((16,), bias, jnp.float32)
    for d in range(EMB):
      col = plsc.load_gather(acc_v, [idxb, jnp.full((16,), d, jnp.int32)])
      x = col * inv_seq
      e = jnp.exp(jnp.abs(x) * 2.0)
      t = jnp.sign(x) * (1.0 - 2.0 / (e + 1.0))
      w_d = w_lo[d] if d < 16 else w_hi[d - 16]
      o = o + t * w_d
    out_v[pl.ds(g * 16, 16)] = o
    return 0

  lax.fori_loop(0, BPW // 16, ep_body, 0)

  pltpu.sync_copy(out_v, out_hbm.at[pl.ds(base, BPW)])


@jax.jit
def _danmodel(inp, tbl, wb):
  mesh = plsc.VectorSubcoreMesh(
      core_axis_name="c", subcore_axis_name="s", num_cores=NC,
      num_subcores=NS)
  return pl.kernel(
      _danmodel_body,
      out_type=jax.ShapeDtypeStruct((BATCH,), jnp.float32),
      mesh=mesh,
      compiler_params=pltpu.CompilerParams(
          needs_layout_passes=False, use_tc_tiling_on_sc=False),
      scratch_types=[
          pltpu.VMEM((SEQ, BPW), jnp.int32),
          pltpu.VMEM((NSLOT * BPW, EMB), jnp.float32),
          pltpu.VMEM((BPW, EMB), jnp.float32),
          pltpu.VMEM((64,), jnp.float32),
          pltpu.VMEM((BPW,), jnp.float32),
          pltpu.SemaphoreType.DMA,
          pltpu.SemaphoreType.DMA,
      ],
  )(inp, tbl, wb)


def kernel(input, emb_table, W, b):
  inp = input.astype(jnp.int32)
  wb = jnp.concatenate(
      [W.reshape(-1), b.reshape(-1),
       jnp.zeros((64 - EMB - 1,), jnp.float32)]).astype(jnp.float32)
  out = _danmodel(inp, emb_table, wb)
  return out.reshape(BATCH, 1)
